# baseline (speedup 1.0000x reference)
import jax
import jax.numpy as jnp
from jax.experimental import pallas as pl

N = 100000
L = 3


def _copy_body(x_ref, o_ref):
    o_ref[...] = x_ref[...]


def kernel(x, h, edge_attr, edge_index, mlp_x_w0, mlp_x_b0, mlp_x_w1, mlp_x_b1,
           mlp_h_w0, mlp_h_b0, mlp_h_w1, mlp_h_b1,
           gat_W, gat_att_src, gat_att_dst, gat_lin_edge, gat_att_edge, gat_bias,
           fc_mu_w0, fc_mu_b0, fc_mu_w1, fc_mu_b1):
    silu = jax.nn.silu
    mlp_x_w0 = pl.pallas_call(
        _copy_body, out_shape=jax.ShapeDtypeStruct(mlp_x_w0.shape, mlp_x_w0.dtype))(mlp_x_w0)
    x_l = silu(x @ mlp_x_w0 + mlp_x_b0) @ mlp_x_w1 + mlp_x_b1
    h_l = silu(h @ mlp_h_w0 + mlp_h_b0) @ mlp_h_w1 + mlp_h_b1
    inp = jnp.concatenate([x_l, h_l], axis=-1)
    loop = jnp.arange(N, dtype=edge_index.dtype)
    src = jnp.concatenate([edge_index[0], loop])
    dst = jnp.concatenate([edge_index[1], loop])
    ea_mean = jnp.mean(edge_attr, axis=0, keepdims=True)
    ea = jnp.concatenate([edge_attr, jnp.broadcast_to(ea_mean, (N, edge_attr.shape[1]))], axis=0)
    for ii in range(L):
        xl = inp @ gat_W[ii]
        a_src = (xl * gat_att_src[ii]).sum(axis=-1)
        a_dst = (xl * gat_att_dst[ii]).sum(axis=-1)
        a_edge = ((ea @ gat_lin_edge[ii]) * gat_att_edge[ii]).sum(axis=-1)
        alpha = a_src[src] + a_dst[dst] + a_edge
        alpha = jax.nn.leaky_relu(alpha, 0.2)
        amax = jax.ops.segment_max(alpha, dst, num_segments=N)
        amax = jnp.where(jnp.isfinite(amax), amax, 0.0)
        ex = jnp.exp(alpha - amax[dst])
        denom = jax.ops.segment_sum(ex, dst, num_segments=N)
        coef = ex / (denom[dst] + 1e-16)
        inp = jax.ops.segment_sum(xl[src] * coef[:, None], dst, num_segments=N) + gat_bias[ii]
    mu = silu(inp @ fc_mu_w0 + fc_mu_b0) @ fc_mu_w1 + fc_mu_b1
    return mu.reshape(N, 8, 3)


# speedup vs baseline: 15.1387x; 15.1387x over previous
import functools

import jax
import jax.numpy as jnp
from jax import lax
from jax.experimental import pallas as pl
from jax.experimental.pallas import tpu as pltpu
from jax.experimental.pallas import tpu_sc as plsc

NN = 100000
EE = 1600000
DD = 64
NLAYER = 3

NCORE = 1
NSUB = 16
PER_TILE = 6400
NP = NSUB * PER_TILE
ET = EE // (NCORE * NSUB)
KC = 2000
NCH = ET // KC
KC2 = 800
NCH2 = ET // KC2
RB = 800
GRID = NN // RB

_f32 = jnp.float32


def _ea_stats_body(ea_ref, out_ref):
    blk = ea_ref[...]
    out_ref[0] = jnp.sum(blk)
    out_ref[1] = jnp.max(blk)
    out_ref[2] = jnp.min(blk)
    out_ref[3] = 0.0


def _ea_stats(ea):
    return pl.pallas_call(
        _ea_stats_body,
        out_shape=jax.ShapeDtypeStruct((4,), _f32),
        out_specs=pl.BlockSpec(memory_space=pltpu.SMEM),
    )(ea.reshape(3125, 512))


def _encode_body(x_ref, h_ref, xw0, xb0, xw1, xb1, hw0, hb0, hw1, hb1, o_ref):
    silu = jax.nn.silu
    xl = silu(jnp.dot(x_ref[...], xw0[...], preferred_element_type=_f32)
              + xb0[...]) @ xw1[...] + xb1[...]
    hl = silu(jnp.dot(h_ref[...], hw0[...], preferred_element_type=_f32)
              + hb0[...]) @ hw1[...] + hb1[...]
    o_ref[...] = jnp.concatenate([xl, hl], axis=1)


def _encode(x, h, xw0, xb0, xw1, xb1, hw0, hb0, hw1, hb1):
    R = 2000
    full = lambda a: pl.BlockSpec(a.shape, lambda i: (0,) * a.ndim)
    return pl.pallas_call(
        _encode_body,
        grid=(NN // R,),
        in_specs=[pl.BlockSpec((R, 3), lambda i: (i, 0)),
                  pl.BlockSpec((R, 5), lambda i: (i, 0)),
                  full(xw0), full(xb0), full(xw1), full(xb1),
                  full(hw0), full(hb0), full(hw1), full(hb1)],
        out_specs=pl.BlockSpec((R, DD), lambda i: (i, 0)),
        out_shape=jax.ShapeDtypeStruct((NN, DD), _f32),
    )(x, h, xw0, xb0, xw1, xb1, hw0, hb0, hw1, hb1)


def _prep_body(nparts, *refs):
    it = iter(refs)
    if nparts:
        oinit = next(it)
        parts = [next(it) for _ in range(nparts)]
    else:
        inp_ref = next(it)
    w_ref, asrc_ref, adst_ref, lin_ref, ae_ref, stats_ref = (next(it) for _ in range(6))
    xl_ref, x0_ref, x1_ref, x2_ref, x3_ref, a3_ref, b3_ref, scal_ref = (
        next(it) for _ in range(8))
    acc = next(it)

    i = pl.program_id(0)
    if nparts:
        cols = []
        for r in range(4):
            col = parts[r][...]
            for c in range(1, NCORE):
                col = col + parts[c * 4 + r][...]
            cols.append(col)
        inpb = oinit[...] + jnp.concatenate(cols, axis=1)
    else:
        inpb = inp_ref[...]
    xlb = jnp.dot(inpb, w_ref[...], preferred_element_type=_f32)
    xl_ref[...] = xlb
    x0_ref[...] = xlb[:, 0:16]
    x1_ref[...] = xlb[:, 16:32]
    x2_ref[...] = xlb[:, 32:48]
    x3_ref[...] = xlb[:, 48:64]
    asb = jnp.sum(xlb * asrc_ref[...], axis=1)
    adb = jnp.sum(xlb * adst_ref[...], axis=1)
    a3_ref[0, 0, :] = asb
    b3_ref[0, 0, :] = adb
    m1 = jnp.max(asb)
    m2 = jnp.max(adb)

    @pl.when(i == 0)
    def _():
        acc[0] = m1
        acc[1] = m2

    @pl.when(i > 0)
    def _():
        acc[0] = jnp.maximum(acc[0], m1)
        acc[1] = jnp.maximum(acc[1], m2)

    @pl.when(i == GRID - 1)
    def _():
        c = jnp.sum(lin_ref[...] * ae_ref[...])
        eam = stats_ref[0] / EE
        am = jnp.maximum(jnp.maximum(c * stats_ref[1], c * stats_ref[2]), c * eam)
        sraw = acc[0] + acc[1] + am
        scal_ref[0] = c
        scal_ref[1] = jnp.where(sraw >= 0.0, sraw, 0.2 * sraw)
        scal_ref[2] = eam


def _prep(oinit, parts, W, asrc_row, adst_row, lin_row, ae_row, stats):
    nparts = 0 if parts is None else len(parts)
    full = lambda a: pl.BlockSpec(a.shape, lambda i: (0,) * a.ndim)
    in_specs = [pl.BlockSpec((RB, DD), lambda i: (i, 0))]
    args = [oinit]
    if nparts:
        in_specs += [pl.BlockSpec((RB, 16), lambda i: (i, 0))] * nparts
        args += list(parts)
    in_specs += [full(W), full(asrc_row), full(adst_row), full(lin_row),
                 full(ae_row), pl.BlockSpec(memory_space=pltpu.SMEM)]
    args += [W, asrc_row, adst_row, lin_row, ae_row, stats]
    out_specs = [
        pl.BlockSpec((RB, DD), lambda i: (i, 0)),
        pl.BlockSpec((RB, 16), lambda i: (i, 0)),
        pl.BlockSpec((RB, 16), lambda i: (i, 0)),
        pl.BlockSpec((RB, 16), lambda i: (i, 0)),
        pl.BlockSpec((RB, 16), lambda i: (i, 0)),
        pl.BlockSpec((1, 1, RB), lambda i: (i, 0, 0)),
        pl.BlockSpec((1, 1, RB), lambda i: (i, 0, 0)),
        pl.BlockSpec(memory_space=pltpu.SMEM),
    ]
    out_shape = [
        jax.ShapeDtypeStruct((NN, DD), _f32),
        jax.ShapeDtypeStruct((NN, 16), _f32),
        jax.ShapeDtypeStruct((NN, 16), _f32),
        jax.ShapeDtypeStruct((NN, 16), _f32),
        jax.ShapeDtypeStruct((NN, 16), _f32),
        jax.ShapeDtypeStruct((GRID, 1, RB), _f32),
        jax.ShapeDtypeStruct((GRID, 1, RB), _f32),
        jax.ShapeDtypeStruct((16,), _f32),
    ]
    return pl.pallas_call(
        functools.partial(_prep_body, nparts),
        grid=(GRID,),
        in_specs=in_specs,
        out_specs=out_specs,
        out_shape=out_shape,
        scratch_shapes=[pltpu.SMEM((4,), _f32)],
    )(*args)


def _combine_body(dpt_ref, a3_ref, b3_ref, xl_ref, bias_ref, scal_ref,
                  oinit_ref, rd3_ref):
    p = dpt_ref[0]
    de = p[0, :]
    for c in range(1, NCORE):
        de = de + p[c, :]
    asb = a3_ref[0, 0, :]
    adb = b3_ref[0, 0, :]
    c = scal_ref[0]
    S = scal_ref[1]
    eam = scal_ref[2]
    t = asb + adb + c * eam
    t = jnp.where(t >= 0.0, t, 0.2 * t)
    exs = jnp.exp(t - S)
    rden = 1.0 / (de + exs + 1e-16)
    rd3_ref[0, 0, :] = rden
    cs = exs * rden
    oinit_ref[...] = cs[:, None] * xl_ref[...] + bias_ref[...]


def _combine(dpt, a3, b3, xl, bias_row, scal):
    return pl.pallas_call(
        _combine_body,
        grid=(GRID,),
        in_specs=[pl.BlockSpec((1, NCORE, RB), lambda i: (i, 0, 0)),
                  pl.BlockSpec((1, 1, RB), lambda i: (i, 0, 0)),
                  pl.BlockSpec((1, 1, RB), lambda i: (i, 0, 0)),
                  pl.BlockSpec((RB, DD), lambda i: (i, 0)),
                  pl.BlockSpec((1, DD), lambda i: (0, 0)),
                  pl.BlockSpec(memory_space=pltpu.SMEM)],
        out_specs=[pl.BlockSpec((RB, DD), lambda i: (i, 0)),
                   pl.BlockSpec((1, 1, RB), lambda i: (i, 0, 0))],
        out_shape=[jax.ShapeDtypeStruct((NN, DD), _f32),
                   jax.ShapeDtypeStruct((GRID, 1, RB), _f32)],
    )(dpt, a3, b3, xl, bias_row, scal)


def _decode_body(oinit_ref, *refs):
    parts = refs[:4 * NCORE]
    w0_ref, b0_ref, w1_ref, b1_ref, o_ref = refs[4 * NCORE:]
    cols = []
    for r in range(4):
        col = parts[r][...]
        for c in range(1, NCORE):
            col = col + parts[c * 4 + r][...]
        cols.append(col)
    inpb = oinit_ref[...] + jnp.concatenate(cols, axis=1)
    hh = jax.nn.silu(jnp.dot(inpb, w0_ref[...], preferred_element_type=_f32)
                     + b0_ref[...])
    o_ref[...] = jnp.dot(hh, w1_ref[...], preferred_element_type=_f32) + b1_ref[...]


def _decode(oinit, parts, w0, b0_row, w1, b1_row):
    full = lambda a: pl.BlockSpec(a.shape, lambda i: (0,) * a.ndim)
    return pl.pallas_call(
        _decode_body,
        grid=(GRID,),
        in_specs=[pl.BlockSpec((RB, DD), lambda i: (i, 0))]
                 + [pl.BlockSpec((RB, 16), lambda i: (i, 0))] * (4 * NCORE)
                 + [full(w0), full(b0_row), full(w1), full(b1_row)],
        out_specs=pl.BlockSpec((RB, 24), lambda i: (i, 0)),
        out_shape=jax.ShapeDtypeStruct((NN, 24), _f32),
    )(oinit, *parts, w0, b0_row, w1, b1_row)


_MESH = plsc.VectorSubcoreMesh(core_axis_name="c", subcore_axis_name="s",
                               num_cores=1, num_subcores=16)


def _sc1_body(src_h, dst_h, ea_h, asrc_h, adst_h, scal_h,
              ex_h, dpart_h,
              si, di, eab, asg, adg, exb, zb, scv, dslab, sem):
    cid = lax.axis_index("c")
    sid = lax.axis_index("s")
    pltpu.sync_copy(scal_h, scv)
    sv = scv[pl.ds(0, 16)]
    c = sv[0]
    S = sv[1]

    def zfill(i, carry):
        zb[pl.ds(i * 16, 16)] = jnp.zeros((16,), _f32)
        return carry

    lax.fori_loop(0, PER_TILE // 16, zfill, 0)
    pltpu.sync_copy(zb, dslab.at[pl.ds(sid * PER_TILE, PER_TILE)])
    plsc.subcore_barrier()

    base_e = cid * (EE // NCORE) + sid * ET

    def chunk(j, carry):
        off = base_e + j * KC
        pltpu.sync_copy(src_h.at[pl.ds(off, KC)], si)
        pltpu.sync_copy(dst_h.at[pl.ds(off, KC)], di)
        pltpu.sync_copy(ea_h.at[pl.ds(off, KC)], eab)
        pltpu.async_copy(asrc_h.at[si], asg, sem).wait()
        pltpu.async_copy(adst_h.at[di], adg, sem).wait()

        def comp(t, carry2):
            sl = pl.ds(t * 16, 16)
            av = asg[sl] + adg[sl] + c * eab[sl]
            av = jnp.where(av >= 0.0, av, 0.2 * av) - S
            exb[sl] = jnp.exp(av)
            return carry2

        lax.fori_loop(0, KC // 16, comp, 0)
        pltpu.sync_copy(exb, ex_h.at[pl.ds(off, KC)])
        pltpu.sync_copy(exb, dslab.at[di], add=True)
        return carry

    lax.fori_loop(0, NCH, chunk, 0)
    plsc.subcore_barrier()
    pltpu.sync_copy(dslab.at[pl.ds(sid * PER_TILE, PER_TILE)], zb)
    pltpu.sync_copy(zb, dpart_h.at[pl.ds(cid * NP + sid * PER_TILE, PER_TILE)])


_sc1 = pl.kernel(
    _sc1_body,
    out_type=[jax.ShapeDtypeStruct((EE,), _f32),
              jax.ShapeDtypeStruct((NCORE * NP,), _f32)],
    mesh=_MESH,
    scratch_types=[pltpu.VMEM((KC,), jnp.int32),
                   pltpu.VMEM((KC,), jnp.int32),
                   pltpu.VMEM((KC,), _f32),
                   pltpu.VMEM((KC,), _f32),
                   pltpu.VMEM((KC,), _f32),
                   pltpu.VMEM((KC,), _f32),
                   pltpu.VMEM((PER_TILE,), _f32),
                   pltpu.VMEM((16,), _f32),
                   pltpu.VMEM_SHARED((NP,), _f32),
                   pltpu.SemaphoreType.DMA],
    compiler_params=pltpu.CompilerParams(use_tc_tiling_on_sc=False),
)


def _sc2_body(src_h, dst_h, ex_h, rd_h, x0_h, x1_h, x2_h, x3_h,
              coef_h, part_h,
              si, di, exb, rdg, cfb, rows, slab, sem):
    cid = lax.axis_index("c")
    sid = lax.axis_index("s")

    def zero_rows():
        def zf(i, carry):
            rows[i, :] = jnp.zeros((16,), _f32)
            return carry
        lax.fori_loop(0, 800, zf, 0)

    def zero_slab():
        def zc(i, carry):
            pltpu.sync_copy(rows.at[pl.ds(0, 800), :],
                            slab.at[pl.ds(sid * PER_TILE + i * 800, 800), :])
            return carry
        lax.fori_loop(0, 8, zc, 0)

    base_e = cid * (EE // NCORE) + sid * ET
    zero_rows()
    zero_slab()
    plsc.subcore_barrier()

    for r in range(4):
        xr_h = (x0_h, x1_h, x2_h, x3_h)[r]

        def chunk(j, carry, xr_h=xr_h, r=r):
            off = base_e + j * KC2
            pltpu.sync_copy(src_h.at[pl.ds(off, KC2)], si)
            pltpu.sync_copy(dst_h.at[pl.ds(off, KC2)], di)
            if r == 0:
                pltpu.sync_copy(ex_h.at[pl.ds(off, KC2)], exb)
                pltpu.async_copy(rd_h.at[di], rdg, sem).wait()

                def cf(t, carry2):
                    sl = pl.ds(t * 16, 16)
                    cfb[sl] = exb[sl] * rdg[sl]
                    return carry2

                lax.fori_loop(0, KC2 // 16, cf, 0)
                pltpu.sync_copy(cfb, coef_h.at[pl.ds(off, KC2)])
            else:
                pltpu.sync_copy(coef_h.at[pl.ds(off, KC2)], cfb)
            pltpu.async_copy(xr_h.at[si], rows, sem).wait()

            def scale(t, carry2):
                cv = cfb[pl.ds(t * 16, 16)]
                for jj in range(16):
                    rows[t * 16 + jj, :] = rows[t * 16 + jj, :] * cv[jj]
                return carry2

            lax.fori_loop(0, KC2 // 16, scale, 0)
            pltpu.sync_copy(rows, slab.at[di], add=True)
            return carry

        lax.fori_loop(0, NCH2, chunk, 0)
        plsc.subcore_barrier()

        def cpout(i, carry, r=r):
            pltpu.sync_copy(
                slab.at[pl.ds(sid * PER_TILE + i * 800, 800), :],
                rows.at[pl.ds(0, 800), :])
            pltpu.sync_copy(
                rows.at[pl.ds(0, 800), :],
                part_h.at[pl.ds((cid * 4 + r) * NP + sid * PER_TILE + i * 800,
                                800), :])
            return carry

        lax.fori_loop(0, 8, cpout, 0)
        if r < 3:
            zero_rows()
            zero_slab()
        plsc.subcore_barrier()


_sc2 = pl.kernel(
    _sc2_body,
    out_type=[jax.ShapeDtypeStruct((EE,), _f32),
              jax.ShapeDtypeStruct((NCORE * 4 * NP, 16), _f32)],
    mesh=_MESH,
    scratch_types=[pltpu.VMEM((KC2,), jnp.int32),
                   pltpu.VMEM((KC2,), jnp.int32),
                   pltpu.VMEM((KC2,), _f32),
                   pltpu.VMEM((KC2,), _f32),
                   pltpu.VMEM((KC2,), _f32),
                   pltpu.VMEM((KC2, 16), _f32),
                   pltpu.VMEM_SHARED((NP, 16), _f32),
                   pltpu.SemaphoreType.DMA],
    compiler_params=pltpu.CompilerParams(use_tc_tiling_on_sc=False),
)


def kernel(x, h, edge_attr, edge_index, mlp_x_w0, mlp_x_b0, mlp_x_w1, mlp_x_b1,
           mlp_h_w0, mlp_h_b0, mlp_h_w1, mlp_h_b1,
           gat_W, gat_att_src, gat_att_dst, gat_lin_edge, gat_att_edge, gat_bias,
           fc_mu_w0, fc_mu_b0, fc_mu_w1, fc_mu_b1):
    src = edge_index[0]
    dst = edge_index[1]
    ea = edge_attr[:, 0]
    row = lambda v: v.reshape(1, -1)

    stats = _ea_stats(ea)
    oinit = _encode(x, h, mlp_x_w0, row(mlp_x_b0), mlp_x_w1, row(mlp_x_b1),
                    mlp_h_w0, row(mlp_h_b0), mlp_h_w1, row(mlp_h_b1))
    parts = None
    for ii in range(NLAYER):
        xl, x0, x1, x2, x3, a3, b3, scal = _prep(
            oinit, parts, gat_W[ii], row(gat_att_src[ii]), row(gat_att_dst[ii]),
            gat_lin_edge[ii], row(gat_att_edge[ii]), stats)
        ex, dpart = _sc1(src, dst, ea, a3.reshape(NN), b3.reshape(NN), scal)
        dpt = (dpart.reshape(NCORE, NP)[:, :NN]
               .reshape(NCORE, GRID, RB).transpose(1, 0, 2))
        oinit, rd3 = _combine(dpt, a3, b3, xl, row(gat_bias[ii]), scal)
        _, part = _sc2(src, dst, ex, rd3.reshape(NN), x0, x1, x2, x3)
        pr = part.reshape(NCORE * 4, NP, 16)
        parts = tuple(pr[i] for i in range(NCORE * 4))

    mu = _decode(oinit, parts, fc_mu_w0, row(fc_mu_b0), fc_mu_w1, row(fc_mu_b1))
    return mu.reshape(NN, 8, 3)


# speedup vs baseline: 18.8801x; 1.2471x over previous
import functools

import jax
import jax.numpy as jnp
from jax import lax
from jax.experimental import pallas as pl
from jax.experimental.pallas import tpu as pltpu
from jax.experimental.pallas import tpu_sc as plsc

NN = 100000
EE = 1600000
DD = 64
NLAYER = 3

NCORE = 1
NSUB = 16
PER_TILE = 6400
NP = NSUB * PER_TILE
ET = EE // (NCORE * NSUB)
KC = 2000
NCH = ET // KC
KC2 = 400
NCH2 = ET // KC2
RB = 800
GRID = NN // RB

_f32 = jnp.float32


def _ea_stats_body(ea_ref, out_ref):
    blk = ea_ref[...]
    out_ref[0] = jnp.sum(blk)
    out_ref[1] = jnp.max(blk)
    out_ref[2] = jnp.min(blk)
    out_ref[3] = 0.0


def _ea_stats(ea):
    return pl.pallas_call(
        _ea_stats_body,
        out_shape=jax.ShapeDtypeStruct((4,), _f32),
        out_specs=pl.BlockSpec(memory_space=pltpu.SMEM),
    )(ea.reshape(3125, 512))


def _encode_body(x_ref, h_ref, xw0, xb0, xw1, xb1, hw0, hb0, hw1, hb1, o_ref):
    silu = jax.nn.silu
    xl = silu(jnp.dot(x_ref[...], xw0[...], preferred_element_type=_f32)
              + xb0[...]) @ xw1[...] + xb1[...]
    hl = silu(jnp.dot(h_ref[...], hw0[...], preferred_element_type=_f32)
              + hb0[...]) @ hw1[...] + hb1[...]
    o_ref[...] = jnp.concatenate([xl, hl], axis=1)


def _encode(x, h, xw0, xb0, xw1, xb1, hw0, hb0, hw1, hb1):
    R = 2000
    full = lambda a: pl.BlockSpec(a.shape, lambda i: (0,) * a.ndim)
    return pl.pallas_call(
        _encode_body,
        grid=(NN // R,),
        in_specs=[pl.BlockSpec((R, 3), lambda i: (i, 0)),
                  pl.BlockSpec((R, 5), lambda i: (i, 0)),
                  full(xw0), full(xb0), full(xw1), full(xb1),
                  full(hw0), full(hb0), full(hw1), full(hb1)],
        out_specs=pl.BlockSpec((R, DD), lambda i: (i, 0)),
        out_shape=jax.ShapeDtypeStruct((NN, DD), _f32),
    )(x, h, xw0, xb0, xw1, xb1, hw0, hb0, hw1, hb1)


def _prep_body(nparts, *refs):
    it = iter(refs)
    if nparts:
        oinit = next(it)
        parts = [next(it) for _ in range(nparts)]
    else:
        inp_ref = next(it)
    w_ref, asrc_ref, adst_ref, lin_ref, ae_ref, stats_ref = (next(it) for _ in range(6))
    xl_ref, x0_ref, x1_ref, x2_ref, x3_ref, a3_ref, b3_ref, scal_ref = (
        next(it) for _ in range(8))
    acc = next(it)

    i = pl.program_id(0)
    if nparts:
        cols = []
        for r in range(4):
            col = parts[r][...]
            for c in range(1, NCORE):
                col = col + parts[c * 4 + r][...]
            cols.append(col)
        inpb = oinit[...] + jnp.concatenate(cols, axis=1)
    else:
        inpb = inp_ref[...]
    xlb = jnp.dot(inpb, w_ref[...], preferred_element_type=_f32)
    xl_ref[...] = xlb
    x0_ref[...] = xlb[:, 0:16]
    x1_ref[...] = xlb[:, 16:32]
    x2_ref[...] = xlb[:, 32:48]
    x3_ref[...] = xlb[:, 48:64]
    asb = jnp.sum(xlb * asrc_ref[...], axis=1)
    adb = jnp.sum(xlb * adst_ref[...], axis=1)
    a3_ref[0, 0, :] = asb
    b3_ref[0, 0, :] = adb
    m1 = jnp.max(asb)
    m2 = jnp.max(adb)

    @pl.when(i == 0)
    def _():
        acc[0] = m1
        acc[1] = m2

    @pl.when(i > 0)
    def _():
        acc[0] = jnp.maximum(acc[0], m1)
        acc[1] = jnp.maximum(acc[1], m2)

    @pl.when(i == GRID - 1)
    def _():
        c = jnp.sum(lin_ref[...] * ae_ref[...])
        eam = stats_ref[0] / EE
        am = jnp.maximum(jnp.maximum(c * stats_ref[1], c * stats_ref[2]), c * eam)
        sraw = acc[0] + acc[1] + am
        scal_ref[0] = c
        scal_ref[1] = jnp.where(sraw >= 0.0, sraw, 0.2 * sraw)
        scal_ref[2] = eam


def _prep(oinit, parts, W, asrc_row, adst_row, lin_row, ae_row, stats):
    nparts = 0 if parts is None else len(parts)
    full = lambda a: pl.BlockSpec(a.shape, lambda i: (0,) * a.ndim)
    in_specs = [pl.BlockSpec((RB, DD), lambda i: (i, 0))]
    args = [oinit]
    if nparts:
        in_specs += [pl.BlockSpec((RB, 16), lambda i: (i, 0))] * nparts
        args += list(parts)
    in_specs += [full(W), full(asrc_row), full(adst_row), full(lin_row),
                 full(ae_row), pl.BlockSpec(memory_space=pltpu.SMEM)]
    args += [W, asrc_row, adst_row, lin_row, ae_row, stats]
    out_specs = [
        pl.BlockSpec((RB, DD), lambda i: (i, 0)),
        pl.BlockSpec((RB, 16), lambda i: (i, 0)),
        pl.BlockSpec((RB, 16), lambda i: (i, 0)),
        pl.BlockSpec((RB, 16), lambda i: (i, 0)),
        pl.BlockSpec((RB, 16), lambda i: (i, 0)),
        pl.BlockSpec((1, 1, RB), lambda i: (i, 0, 0)),
        pl.BlockSpec((1, 1, RB), lambda i: (i, 0, 0)),
        pl.BlockSpec(memory_space=pltpu.SMEM),
    ]
    out_shape = [
        jax.ShapeDtypeStruct((NN, DD), _f32),
        jax.ShapeDtypeStruct((NN, 16), _f32),
        jax.ShapeDtypeStruct((NN, 16), _f32),
        jax.ShapeDtypeStruct((NN, 16), _f32),
        jax.ShapeDtypeStruct((NN, 16), _f32),
        jax.ShapeDtypeStruct((GRID, 1, RB), _f32),
        jax.ShapeDtypeStruct((GRID, 1, RB), _f32),
        jax.ShapeDtypeStruct((16,), _f32),
    ]
    return pl.pallas_call(
        functools.partial(_prep_body, nparts),
        grid=(GRID,),
        in_specs=in_specs,
        out_specs=out_specs,
        out_shape=out_shape,
        scratch_shapes=[pltpu.SMEM((4,), _f32)],
    )(*args)


def _combine_body(dpt_ref, a3_ref, b3_ref, xl_ref, bias_ref, scal_ref,
                  oinit_ref, rd3_ref):
    p = dpt_ref[0]
    de = p[0, :]
    for c in range(1, NCORE):
        de = de + p[c, :]
    asb = a3_ref[0, 0, :]
    adb = b3_ref[0, 0, :]
    c = scal_ref[0]
    S = scal_ref[1]
    eam = scal_ref[2]
    t = asb + adb + c * eam
    t = jnp.where(t >= 0.0, t, 0.2 * t)
    exs = jnp.exp(t - S)
    rden = 1.0 / (de + exs + 1e-16)
    rd3_ref[0, 0, :] = rden
    cs = exs * rden
    oinit_ref[...] = cs[:, None] * xl_ref[...] + bias_ref[...]


def _combine(dpt, a3, b3, xl, bias_row, scal):
    return pl.pallas_call(
        _combine_body,
        grid=(GRID,),
        in_specs=[pl.BlockSpec((1, NCORE, RB), lambda i: (i, 0, 0)),
                  pl.BlockSpec((1, 1, RB), lambda i: (i, 0, 0)),
                  pl.BlockSpec((1, 1, RB), lambda i: (i, 0, 0)),
                  pl.BlockSpec((RB, DD), lambda i: (i, 0)),
                  pl.BlockSpec((1, DD), lambda i: (0, 0)),
                  pl.BlockSpec(memory_space=pltpu.SMEM)],
        out_specs=[pl.BlockSpec((RB, DD), lambda i: (i, 0)),
                   pl.BlockSpec((1, 1, RB), lambda i: (i, 0, 0))],
        out_shape=[jax.ShapeDtypeStruct((NN, DD), _f32),
                   jax.ShapeDtypeStruct((GRID, 1, RB), _f32)],
    )(dpt, a3, b3, xl, bias_row, scal)


def _decode_body(oinit_ref, *refs):
    parts = refs[:4 * NCORE]
    w0_ref, b0_ref, w1_ref, b1_ref, o_ref = refs[4 * NCORE:]
    cols = []
    for r in range(4):
        col = parts[r][...]
        for c in range(1, NCORE):
            col = col + parts[c * 4 + r][...]
        cols.append(col)
    inpb = oinit_ref[...] + jnp.concatenate(cols, axis=1)
    hh = jax.nn.silu(jnp.dot(inpb, w0_ref[...], preferred_element_type=_f32)
                     + b0_ref[...])
    o_ref[...] = jnp.dot(hh, w1_ref[...], preferred_element_type=_f32) + b1_ref[...]


def _decode(oinit, parts, w0, b0_row, w1, b1_row):
    full = lambda a: pl.BlockSpec(a.shape, lambda i: (0,) * a.ndim)
    return pl.pallas_call(
        _decode_body,
        grid=(GRID,),
        in_specs=[pl.BlockSpec((RB, DD), lambda i: (i, 0))]
                 + [pl.BlockSpec((RB, 16), lambda i: (i, 0))] * (4 * NCORE)
                 + [full(w0), full(b0_row), full(w1), full(b1_row)],
        out_specs=pl.BlockSpec((RB, 24), lambda i: (i, 0)),
        out_shape=jax.ShapeDtypeStruct((NN, 24), _f32),
    )(oinit, *parts, w0, b0_row, w1, b1_row)


_MESH = plsc.VectorSubcoreMesh(core_axis_name="c", subcore_axis_name="s",
                               num_cores=1, num_subcores=16)


def _sc1_body(src_h, dst_h, ea_h, asrc_h, adst_h, scal_h,
              ex_h, dpart_h,
              si, di, eab, asg, adg, exb, zb, scv, dslab, sem):
    cid = lax.axis_index("c")
    sid = lax.axis_index("s")
    pltpu.sync_copy(scal_h, scv)
    sv = scv[pl.ds(0, 16)]
    c = sv[0]
    S = sv[1]

    def zfill(i, carry):
        zb[pl.ds(i * 16, 16)] = jnp.zeros((16,), _f32)
        return carry

    lax.fori_loop(0, PER_TILE // 16, zfill, 0)
    pltpu.sync_copy(zb, dslab.at[pl.ds(sid * PER_TILE, PER_TILE)])
    plsc.subcore_barrier()

    base_e = cid * (EE // NCORE) + sid * ET

    def chunk(j, carry):
        off = base_e + j * KC
        pltpu.sync_copy(src_h.at[pl.ds(off, KC)], si)
        pltpu.sync_copy(dst_h.at[pl.ds(off, KC)], di)
        pltpu.sync_copy(ea_h.at[pl.ds(off, KC)], eab)
        pltpu.async_copy(asrc_h.at[si], asg, sem).wait()
        pltpu.async_copy(adst_h.at[di], adg, sem).wait()

        def comp(t, carry2):
            sl = pl.ds(t * 16, 16)
            av = asg[sl] + adg[sl] + c * eab[sl]
            av = jnp.where(av >= 0.0, av, 0.2 * av) - S
            exb[sl] = jnp.exp(av)
            return carry2

        lax.fori_loop(0, KC // 16, comp, 0)
        pltpu.sync_copy(exb, ex_h.at[pl.ds(off, KC)])
        pltpu.sync_copy(exb, dslab.at[di], add=True)
        return carry

    lax.fori_loop(0, NCH, chunk, 0)
    plsc.subcore_barrier()
    pltpu.sync_copy(dslab.at[pl.ds(sid * PER_TILE, PER_TILE)], zb)
    pltpu.sync_copy(zb, dpart_h.at[pl.ds(cid * NP + sid * PER_TILE, PER_TILE)])


_sc1 = pl.kernel(
    _sc1_body,
    out_type=[jax.ShapeDtypeStruct((EE,), _f32),
              jax.ShapeDtypeStruct((NCORE * NP,), _f32)],
    mesh=_MESH,
    scratch_types=[pltpu.VMEM((KC,), jnp.int32),
                   pltpu.VMEM((KC,), jnp.int32),
                   pltpu.VMEM((KC,), _f32),
                   pltpu.VMEM((KC,), _f32),
                   pltpu.VMEM((KC,), _f32),
                   pltpu.VMEM((KC,), _f32),
                   pltpu.VMEM((PER_TILE,), _f32),
                   pltpu.VMEM((16,), _f32),
                   pltpu.VMEM_SHARED((NP,), _f32),
                   pltpu.SemaphoreType.DMA],
    compiler_params=pltpu.CompilerParams(use_tc_tiling_on_sc=False),
)


def _sc2_body(src_h, dst_h, ex_h, rd_h, x0_h, x1_h, x2_h, x3_h,
              coef_h, part_h,
              si0, si1, di0, di1, cfb0, cfb1, exb, rdg,
              rows0, rows1, slab,
              sem_i0, sem_i1, sem_g, sem_s0, sem_s1, sem_r):
    cid = lax.axis_index("c")
    sid = lax.axis_index("s")
    si = (si0, si1)
    di = (di0, di1)
    cfb = (cfb0, cfb1)
    rows = (rows0, rows1)
    sem_i = (sem_i0, sem_i1)
    sem_s = (sem_s0, sem_s1)

    def zero_rows():
        def zf(i, carry):
            rows0[i, :] = jnp.zeros((16,), _f32)
            return carry
        lax.fori_loop(0, KC2, zf, 0)

    def zero_slab():
        def zc(i, carry):
            pltpu.sync_copy(rows0,
                            slab.at[pl.ds(sid * PER_TILE + i * KC2, KC2), :])
            return carry
        lax.fori_loop(0, PER_TILE // KC2, zc, 0)

    base_e = cid * (EE // NCORE) + sid * ET
    zero_rows()
    zero_slab()
    plsc.subcore_barrier()

    for r in range(4):
        xr_h = (x0_h, x1_h, x2_h, x3_h)[r]

        def issue_idx(j, b, r=r):
            off = base_e + j * KC2
            pltpu.async_copy(src_h.at[pl.ds(off, KC2)], si[b], sem_i[b])
            if r == 0:
                pltpu.async_copy(ex_h.at[pl.ds(off, KC2)], exb, sem_i[b])
            else:
                pltpu.async_copy(coef_h.at[pl.ds(off, KC2)], cfb[b], sem_i[b])

        def wait_idx(j, b, r=r):
            off = base_e + j * KC2
            pltpu.make_async_copy(src_h.at[pl.ds(off, KC2)], si[b],
                                  sem_i[b]).wait()
            if r == 0:
                pltpu.make_async_copy(ex_h.at[pl.ds(off, KC2)], exb,
                                      sem_i[b]).wait()
            else:
                pltpu.make_async_copy(coef_h.at[pl.ds(off, KC2)], cfb[b],
                                      sem_i[b]).wait()

        issue_idx(0, 0)

        def pair(jj, carry, r=r, xr_h=xr_h):
            for b in range(2):
                j = 2 * jj + b
                off = base_e + j * KC2

                @pl.when(j >= 2)
                def _(b=b):
                    pltpu.make_async_copy(rows[b], slab.at[di[b]],
                                          sem_s[b]).wait()

                wait_idx(j, b)
                g = pltpu.async_copy(xr_h.at[si[b]], rows[b], sem_g)
                pltpu.sync_copy(dst_h.at[pl.ds(off, KC2)], di[b])
                if r == 0:
                    pltpu.async_copy(rd_h.at[di[b]], rdg, sem_r).wait()

                    def cf(t, carry2, b=b):
                        sl = pl.ds(t * 16, 16)
                        cfb[b][sl] = exb[sl] * rdg[sl]
                        return carry2

                    lax.fori_loop(0, KC2 // 16, cf, 0)
                    pltpu.sync_copy(cfb[b], coef_h.at[pl.ds(off, KC2)])

                @pl.when(j + 1 < NCH2)
                def _(j=j, b=b):
                    issue_idx(j + 1, 1 - b)

                g.wait()

                def scale(t, carry2, b=b):
                    cv = cfb[b][pl.ds(t * 16, 16)]
                    for u in range(16):
                        rows[b][t * 16 + u, :] = rows[b][t * 16 + u, :] * cv[u]
                    return carry2

                lax.fori_loop(0, KC2 // 16, scale, 0)
                pltpu.async_copy(rows[b], slab.at[di[b]], sem_s[b], add=True)
            return carry

        lax.fori_loop(0, NCH2 // 2, pair, 0)
        pltpu.make_async_copy(rows[0], slab.at[di[0]], sem_s[0]).wait()
        pltpu.make_async_copy(rows[1], slab.at[di[1]], sem_s[1]).wait()
        plsc.subcore_barrier()

        def cpout(i, carry, r=r):
            pltpu.sync_copy(
                slab.at[pl.ds(sid * PER_TILE + i * KC2, KC2), :],
                rows0)
            pltpu.sync_copy(
                rows0,
                part_h.at[pl.ds((cid * 4 + r) * NP + sid * PER_TILE + i * KC2,
                                KC2), :])
            return carry

        lax.fori_loop(0, PER_TILE // KC2, cpout, 0)
        if r < 3:
            zero_rows()
            zero_slab()
        plsc.subcore_barrier()


_sc2 = pl.kernel(
    _sc2_body,
    out_type=[jax.ShapeDtypeStruct((EE,), _f32),
              jax.ShapeDtypeStruct((NCORE * 4 * NP, 16), _f32)],
    mesh=_MESH,
    scratch_types=[pltpu.VMEM((KC2,), jnp.int32),
                   pltpu.VMEM((KC2,), jnp.int32),
                   pltpu.VMEM((KC2,), jnp.int32),
                   pltpu.VMEM((KC2,), jnp.int32),
                   pltpu.VMEM((KC2,), _f32),
                   pltpu.VMEM((KC2,), _f32),
                   pltpu.VMEM((KC2,), _f32),
                   pltpu.VMEM((KC2,), _f32),
                   pltpu.VMEM((KC2, 16), _f32),
                   pltpu.VMEM((KC2, 16), _f32),
                   pltpu.VMEM_SHARED((NP, 16), _f32),
                   pltpu.SemaphoreType.DMA,
                   pltpu.SemaphoreType.DMA,
                   pltpu.SemaphoreType.DMA,
                   pltpu.SemaphoreType.DMA,
                   pltpu.SemaphoreType.DMA,
                   pltpu.SemaphoreType.DMA],
    compiler_params=pltpu.CompilerParams(use_tc_tiling_on_sc=False),
)


def kernel(x, h, edge_attr, edge_index, mlp_x_w0, mlp_x_b0, mlp_x_w1, mlp_x_b1,
           mlp_h_w0, mlp_h_b0, mlp_h_w1, mlp_h_b1,
           gat_W, gat_att_src, gat_att_dst, gat_lin_edge, gat_att_edge, gat_bias,
           fc_mu_w0, fc_mu_b0, fc_mu_w1, fc_mu_b1):
    src = edge_index[0]
    dst = edge_index[1]
    ea = edge_attr[:, 0]
    row = lambda v: v.reshape(1, -1)

    stats = _ea_stats(ea)
    oinit = _encode(x, h, mlp_x_w0, row(mlp_x_b0), mlp_x_w1, row(mlp_x_b1),
                    mlp_h_w0, row(mlp_h_b0), mlp_h_w1, row(mlp_h_b1))
    parts = None
    for ii in range(NLAYER):
        xl, x0, x1, x2, x3, a3, b3, scal = _prep(
            oinit, parts, gat_W[ii], row(gat_att_src[ii]), row(gat_att_dst[ii]),
            gat_lin_edge[ii], row(gat_att_edge[ii]), stats)
        ex, dpart = _sc1(src, dst, ea, a3.reshape(NN), b3.reshape(NN), scal)
        dpt = (dpart.reshape(NCORE, NP)[:, :NN]
               .reshape(NCORE, GRID, RB).transpose(1, 0, 2))
        oinit, rd3 = _combine(dpt, a3, b3, xl, row(gat_bias[ii]), scal)
        _, part = _sc2(src, dst, ex, rd3.reshape(NN), x0, x1, x2, x3)
        pr = part.reshape(NCORE * 4, NP, 16)
        parts = tuple(pr[i] for i in range(NCORE * 4))

    mu = _decode(oinit, parts, fc_mu_w0, row(fc_mu_b0), fc_mu_w1, row(fc_mu_b1))
    return mu.reshape(NN, 8, 3)


# speedup vs baseline: 30.5456x; 1.6179x over previous
import functools

import jax
import jax.numpy as jnp
from jax import lax
from jax.experimental import pallas as pl
from jax.experimental.pallas import tpu as pltpu
from jax.experimental.pallas import tpu_sc as plsc

NN = 100000
EE = 1600000
DD = 64
NLAYER = 3

NCORE = 1
NSUB = 16
PER_TILE = 6400
NP = NSUB * PER_TILE
ET = EE // (NCORE * NSUB)
KC = 2000
NCH = ET // KC
KC2 = 400
NCH2 = ET // KC2
RB = 800
GRID = NN // RB

_f32 = jnp.float32


def _ea_stats_body(ea_ref, out_ref):
    blk = ea_ref[...]
    out_ref[0] = jnp.sum(blk)
    out_ref[1] = jnp.max(blk)
    out_ref[2] = jnp.min(blk)
    out_ref[3] = 0.0


def _ea_stats(ea):
    return pl.pallas_call(
        _ea_stats_body,
        out_shape=jax.ShapeDtypeStruct((4,), _f32),
        out_specs=pl.BlockSpec(memory_space=pltpu.SMEM),
    )(ea.reshape(3125, 512))


def _encode_body(x_ref, h_ref, xw0, xb0, xw1, xb1, hw0, hb0, hw1, hb1, o_ref):
    silu = jax.nn.silu
    xl = silu(jnp.dot(x_ref[...], xw0[...], preferred_element_type=_f32)
              + xb0[...]) @ xw1[...] + xb1[...]
    hl = silu(jnp.dot(h_ref[...], hw0[...], preferred_element_type=_f32)
              + hb0[...]) @ hw1[...] + hb1[...]
    o_ref[...] = jnp.concatenate([xl, hl], axis=1)


def _encode(x, h, xw0, xb0, xw1, xb1, hw0, hb0, hw1, hb1):
    R = 2000
    full = lambda a: pl.BlockSpec(a.shape, lambda i: (0,) * a.ndim)
    return pl.pallas_call(
        _encode_body,
        grid=(NN // R,),
        in_specs=[pl.BlockSpec((R, 3), lambda i: (i, 0)),
                  pl.BlockSpec((R, 5), lambda i: (i, 0)),
                  full(xw0), full(xb0), full(xw1), full(xb1),
                  full(hw0), full(hb0), full(hw1), full(hb1)],
        out_specs=pl.BlockSpec((R, DD), lambda i: (i, 0)),
        out_shape=jax.ShapeDtypeStruct((NN, DD), _f32),
    )(x, h, xw0, xb0, xw1, xb1, hw0, hb0, hw1, hb1)


def _prep_body(nparts, *refs):
    it = iter(refs)
    if nparts:
        oinit = next(it)
        parts = [next(it) for _ in range(nparts)]
    else:
        inp_ref = next(it)
    w_ref, asrc_ref, adst_ref, lin_ref, ae_ref, stats_ref = (next(it) for _ in range(6))
    xl_ref, x0_ref, x1_ref, x2_ref, x3_ref, a3_ref, b3_ref, scal_ref = (
        next(it) for _ in range(8))
    acc = next(it)

    i = pl.program_id(0)
    if nparts:
        cols = []
        for r in range(4):
            col = parts[r][...]
            for c in range(1, NCORE):
                col = col + parts[c * 4 + r][...]
            cols.append(col)
        inpb = oinit[...] + jnp.concatenate(cols, axis=1)
    else:
        inpb = inp_ref[...]
    xlb = jnp.dot(inpb, w_ref[...], preferred_element_type=_f32)
    xl_ref[...] = xlb
    x0_ref[...] = xlb[:, 0:16]
    x1_ref[...] = xlb[:, 16:32]
    x2_ref[...] = xlb[:, 32:48]
    x3_ref[...] = xlb[:, 48:64]
    asb = jnp.sum(xlb * asrc_ref[...], axis=1)
    adb = jnp.sum(xlb * adst_ref[...], axis=1)
    a3_ref[0, 0, :] = asb
    b3_ref[0, 0, :] = adb
    m1 = jnp.max(asb)
    m2 = jnp.max(adb)

    @pl.when(i == 0)
    def _():
        acc[0] = m1
        acc[1] = m2

    @pl.when(i > 0)
    def _():
        acc[0] = jnp.maximum(acc[0], m1)
        acc[1] = jnp.maximum(acc[1], m2)

    @pl.when(i == GRID - 1)
    def _():
        c = jnp.sum(lin_ref[...] * ae_ref[...])
        eam = stats_ref[0] / EE
        am = jnp.maximum(jnp.maximum(c * stats_ref[1], c * stats_ref[2]), c * eam)
        sraw = acc[0] + acc[1] + am
        scal_ref[0] = c
        scal_ref[1] = jnp.where(sraw >= 0.0, sraw, 0.2 * sraw)
        scal_ref[2] = eam


_NPB = NP // RB


def _prep(oinit, part_flat, W, asrc_row, adst_row, lin_row, ae_row, stats):
    nparts = 0 if part_flat is None else 4 * NCORE
    full = lambda a: pl.BlockSpec(a.shape, lambda i: (0,) * a.ndim)
    in_specs = [pl.BlockSpec((RB, DD), lambda i: (i, 0))]
    args = [oinit]
    if nparts:
        in_specs += [pl.BlockSpec((RB, 16), lambda i, k=k: (k * _NPB + i, 0))
                     for k in range(nparts)]
        args += [part_flat] * nparts
    in_specs += [full(W), full(asrc_row), full(adst_row), full(lin_row),
                 full(ae_row), pl.BlockSpec(memory_space=pltpu.SMEM)]
    args += [W, asrc_row, adst_row, lin_row, ae_row, stats]
    out_specs = [
        pl.BlockSpec((RB, DD), lambda i: (i, 0)),
        pl.BlockSpec((RB, 16), lambda i: (i, 0)),
        pl.BlockSpec((RB, 16), lambda i: (i, 0)),
        pl.BlockSpec((RB, 16), lambda i: (i, 0)),
        pl.BlockSpec((RB, 16), lambda i: (i, 0)),
        pl.BlockSpec((1, 1, RB), lambda i: (i, 0, 0)),
        pl.BlockSpec((1, 1, RB), lambda i: (i, 0, 0)),
        pl.BlockSpec(memory_space=pltpu.SMEM),
    ]
    out_shape = [
        jax.ShapeDtypeStruct((NN, DD), _f32),
        jax.ShapeDtypeStruct((NN, 16), _f32),
        jax.ShapeDtypeStruct((NN, 16), _f32),
        jax.ShapeDtypeStruct((NN, 16), _f32),
        jax.ShapeDtypeStruct((NN, 16), _f32),
        jax.ShapeDtypeStruct((GRID, 1, RB), _f32),
        jax.ShapeDtypeStruct((GRID, 1, RB), _f32),
        jax.ShapeDtypeStruct((16,), _f32),
    ]
    return pl.pallas_call(
        functools.partial(_prep_body, nparts),
        grid=(GRID,),
        in_specs=in_specs,
        out_specs=out_specs,
        out_shape=out_shape,
        scratch_shapes=[pltpu.SMEM((4,), _f32)],
    )(*args)


def _combine_body(dpt_ref, a3_ref, b3_ref, xl_ref, bias_ref, scal_ref,
                  oinit_ref, rd3_ref):
    p = dpt_ref[0]
    de = p[0, :]
    for c in range(1, NCORE):
        de = de + p[c, :]
    asb = a3_ref[0, 0, :]
    adb = b3_ref[0, 0, :]
    c = scal_ref[0]
    S = scal_ref[1]
    eam = scal_ref[2]
    t = asb + adb + c * eam
    t = jnp.where(t >= 0.0, t, 0.2 * t)
    exs = jnp.exp(t - S)
    rden = 1.0 / (de + exs + 1e-16)
    rd3_ref[0, 0, :] = rden
    cs = exs * rden
    oinit_ref[...] = cs[:, None] * xl_ref[...] + bias_ref[...]


def _combine(dpt, a3, b3, xl, bias_row, scal):
    return pl.pallas_call(
        _combine_body,
        grid=(GRID,),
        in_specs=[pl.BlockSpec((1, NCORE, RB), lambda i: (i, 0, 0)),
                  pl.BlockSpec((1, 1, RB), lambda i: (i, 0, 0)),
                  pl.BlockSpec((1, 1, RB), lambda i: (i, 0, 0)),
                  pl.BlockSpec((RB, DD), lambda i: (i, 0)),
                  pl.BlockSpec((1, DD), lambda i: (0, 0)),
                  pl.BlockSpec(memory_space=pltpu.SMEM)],
        out_specs=[pl.BlockSpec((RB, DD), lambda i: (i, 0)),
                   pl.BlockSpec((1, 1, RB), lambda i: (i, 0, 0))],
        out_shape=[jax.ShapeDtypeStruct((NN, DD), _f32),
                   jax.ShapeDtypeStruct((GRID, 1, RB), _f32)],
    )(dpt, a3, b3, xl, bias_row, scal)


def _decode_body(oinit_ref, *refs):
    parts = refs[:4 * NCORE]
    w0_ref, b0_ref, w1_ref, b1_ref, o_ref = refs[4 * NCORE:]
    cols = []
    for r in range(4):
        col = parts[r][...]
        for c in range(1, NCORE):
            col = col + parts[c * 4 + r][...]
        cols.append(col)
    inpb = oinit_ref[...] + jnp.concatenate(cols, axis=1)
    hh = jax.nn.silu(jnp.dot(inpb, w0_ref[...], preferred_element_type=_f32)
                     + b0_ref[...])
    o_ref[...] = jnp.dot(hh, w1_ref[...], preferred_element_type=_f32) + b1_ref[...]


def _decode(oinit, part_flat, w0, b0_row, w1, b1_row):
    full = lambda a: pl.BlockSpec(a.shape, lambda i: (0,) * a.ndim)
    return pl.pallas_call(
        _decode_body,
        grid=(GRID,),
        in_specs=[pl.BlockSpec((RB, DD), lambda i: (i, 0))]
                 + [pl.BlockSpec((RB, 16), lambda i, k=k: (k * _NPB + i, 0))
                    for k in range(4 * NCORE)]
                 + [full(w0), full(b0_row), full(w1), full(b1_row)],
        out_specs=pl.BlockSpec((RB, 24), lambda i: (i, 0)),
        out_shape=jax.ShapeDtypeStruct((NN, 24), _f32),
    )(oinit, *([part_flat] * (4 * NCORE)), w0, b0_row, w1, b1_row)


_MESH = plsc.VectorSubcoreMesh(core_axis_name="c", subcore_axis_name="s",
                               num_cores=1, num_subcores=16)


def _sc1_body(src_h, dst_h, ea_h, asrc_h, adst_h, scal_h,
              ex_h, dpart_h,
              si0, si1, di0, di1, ea0, ea1, ex0, ex1, asg, adg, zb, scv, dslab,
              sem_i0, sem_i1, sem_g, sem_w0, sem_w1, sem_s0, sem_s1):
    cid = lax.axis_index("c")
    sid = lax.axis_index("s")
    si = (si0, si1)
    di = (di0, di1)
    eab = (ea0, ea1)
    exb = (ex0, ex1)
    sem_i = (sem_i0, sem_i1)
    sem_w = (sem_w0, sem_w1)
    sem_s = (sem_s0, sem_s1)
    pltpu.sync_copy(scal_h, scv)
    sv = scv[pl.ds(0, 16)]
    c = sv[0]
    S = sv[1]

    def zfill(i, carry):
        zb[pl.ds(i * 16, 16)] = jnp.zeros((16,), _f32)
        return carry

    lax.fori_loop(0, PER_TILE // 16, zfill, 0)
    pltpu.sync_copy(zb, dslab.at[pl.ds(sid * PER_TILE, PER_TILE)])
    plsc.subcore_barrier()

    base_e = cid * (EE // NCORE) + sid * ET

    def issue_idx(j, b):
        off = base_e + j * KC
        pltpu.async_copy(src_h.at[pl.ds(off, KC)], si[b], sem_i[b])
        pltpu.async_copy(dst_h.at[pl.ds(off, KC)], di[b], sem_i[b])
        pltpu.async_copy(ea_h.at[pl.ds(off, KC)], eab[b], sem_i[b])

    def wait_idx(j, b):
        off = base_e + j * KC
        pltpu.make_async_copy(src_h.at[pl.ds(off, KC)], si[b], sem_i[b]).wait()
        pltpu.make_async_copy(dst_h.at[pl.ds(off, KC)], di[b], sem_i[b]).wait()
        pltpu.make_async_copy(ea_h.at[pl.ds(off, KC)], eab[b], sem_i[b]).wait()

    issue_idx(0, 0)

    def pair(jj, carry):
        for b in range(2):
            j = 2 * jj + b
            off = base_e + j * KC
            wait_idx(j, b)
            g1 = pltpu.async_copy(asrc_h.at[si[b]], asg, sem_g)
            g2 = pltpu.async_copy(adst_h.at[di[b]], adg, sem_g)

            @pl.when(j >= 1)
            def _(b=b, j=j):
                ob = 1 - b
                oo = base_e + (j - 1) * KC
                pltpu.make_async_copy(exb[ob], ex_h.at[pl.ds(oo, KC)],
                                      sem_w[ob]).wait()
                pltpu.make_async_copy(exb[ob], dslab.at[di[ob]],
                                      sem_s[ob]).wait()

            @pl.when(j + 1 < NCH)
            def _(j=j, b=b):
                issue_idx(j + 1, 1 - b)

            g1.wait()
            g2.wait()

            def comp(t, carry2, b=b):
                sl = pl.ds(t * 16, 16)
                av = asg[sl] + adg[sl] + c * eab[b][sl]
                av = jnp.where(av >= 0.0, av, 0.2 * av) - S
                exb[b][sl] = jnp.exp(av)
                return carry2

            lax.fori_loop(0, KC // 16, comp, 0)
            pltpu.async_copy(exb[b], ex_h.at[pl.ds(off, KC)], sem_w[b])
            pltpu.async_copy(exb[b], dslab.at[di[b]], sem_s[b], add=True)
        return carry

    lax.fori_loop(0, NCH // 2, pair, 0)
    lb = 1
    pltpu.make_async_copy(exb[lb], ex_h.at[pl.ds(base_e + (NCH - 1) * KC, KC)],
                          sem_w[lb]).wait()
    pltpu.make_async_copy(exb[lb], dslab.at[di[lb]], sem_s[lb]).wait()
    plsc.subcore_barrier()
    pltpu.sync_copy(dslab.at[pl.ds(sid * PER_TILE, PER_TILE)], zb)
    pltpu.sync_copy(zb, dpart_h.at[pl.ds(cid * NP + sid * PER_TILE, PER_TILE)])


_sc1 = pl.kernel(
    _sc1_body,
    out_type=[jax.ShapeDtypeStruct((EE,), _f32),
              jax.ShapeDtypeStruct((NCORE * NP,), _f32)],
    mesh=_MESH,
    scratch_types=[pltpu.VMEM((KC,), jnp.int32),
                   pltpu.VMEM((KC,), jnp.int32),
                   pltpu.VMEM((KC,), jnp.int32),
                   pltpu.VMEM((KC,), jnp.int32),
                   pltpu.VMEM((KC,), _f32),
                   pltpu.VMEM((KC,), _f32),
                   pltpu.VMEM((KC,), _f32),
                   pltpu.VMEM((KC,), _f32),
                   pltpu.VMEM((KC,), _f32),
                   pltpu.VMEM((KC,), _f32),
                   pltpu.VMEM((PER_TILE,), _f32),
                   pltpu.VMEM((16,), _f32),
                   pltpu.VMEM_SHARED((NP,), _f32),
                   pltpu.SemaphoreType.DMA,
                   pltpu.SemaphoreType.DMA,
                   pltpu.SemaphoreType.DMA,
                   pltpu.SemaphoreType.DMA,
                   pltpu.SemaphoreType.DMA,
                   pltpu.SemaphoreType.DMA,
                   pltpu.SemaphoreType.DMA],
    compiler_params=pltpu.CompilerParams(use_tc_tiling_on_sc=False),
)


def _sc2_body(src_h, dst_h, ex_h, rd_h, x0_h, x1_h, x2_h, x3_h,
              coef_h, part_h,
              si0, si1, di0, di1, cfb0, cfb1, exb, rdg,
              rows0, rows1, slab,
              sem_i0, sem_i1, sem_g, sem_s0, sem_s1, sem_r):
    cid = lax.axis_index("c")
    sid = lax.axis_index("s")
    si = (si0, si1)
    di = (di0, di1)
    cfb = (cfb0, cfb1)
    rows = (rows0, rows1)
    sem_i = (sem_i0, sem_i1)
    sem_s = (sem_s0, sem_s1)

    def zero_rows():
        def zf(i, carry):
            rows0[i, :] = jnp.zeros((16,), _f32)
            return carry
        lax.fori_loop(0, KC2, zf, 0)

    def zero_slab():
        def zc(i, carry):
            pltpu.sync_copy(rows0,
                            slab.at[pl.ds(sid * PER_TILE + i * KC2, KC2), :])
            return carry
        lax.fori_loop(0, PER_TILE // KC2, zc, 0)

    base_e = cid * (EE // NCORE) + sid * ET
    zero_rows()
    zero_slab()
    plsc.subcore_barrier()

    for r in range(4):
        xr_h = (x0_h, x1_h, x2_h, x3_h)[r]

        def issue_idx(j, b, r=r):
            off = base_e + j * KC2
            pltpu.async_copy(src_h.at[pl.ds(off, KC2)], si[b], sem_i[b])
            if r == 0:
                pltpu.async_copy(ex_h.at[pl.ds(off, KC2)], exb, sem_i[b])
            else:
                pltpu.async_copy(coef_h.at[pl.ds(off, KC2)], cfb[b], sem_i[b])

        def wait_idx(j, b, r=r):
            off = base_e + j * KC2
            pltpu.make_async_copy(src_h.at[pl.ds(off, KC2)], si[b],
                                  sem_i[b]).wait()
            if r == 0:
                pltpu.make_async_copy(ex_h.at[pl.ds(off, KC2)], exb,
                                      sem_i[b]).wait()
            else:
                pltpu.make_async_copy(coef_h.at[pl.ds(off, KC2)], cfb[b],
                                      sem_i[b]).wait()

        issue_idx(0, 0)

        def pair(jj, carry, r=r, xr_h=xr_h):
            for b in range(2):
                j = 2 * jj + b
                off = base_e + j * KC2

                @pl.when(j >= 2)
                def _(b=b):
                    pltpu.make_async_copy(rows[b], slab.at[di[b]],
                                          sem_s[b]).wait()

                wait_idx(j, b)
                g = pltpu.async_copy(xr_h.at[si[b]], rows[b], sem_g)
                pltpu.sync_copy(dst_h.at[pl.ds(off, KC2)], di[b])
                if r == 0:
                    pltpu.async_copy(rd_h.at[di[b]], rdg, sem_r).wait()

                    def cf(t, carry2, b=b):
                        sl = pl.ds(t * 16, 16)
                        cfb[b][sl] = exb[sl] * rdg[sl]
                        return carry2

                    lax.fori_loop(0, KC2 // 16, cf, 0)
                    pltpu.sync_copy(cfb[b], coef_h.at[pl.ds(off, KC2)])

                @pl.when(j + 1 < NCH2)
                def _(j=j, b=b):
                    issue_idx(j + 1, 1 - b)

                g.wait()

                def scale(t, carry2, b=b):
                    cv = cfb[b][pl.ds(t * 16, 16)]
                    for u in range(16):
                        rows[b][t * 16 + u, :] = rows[b][t * 16 + u, :] * cv[u]
                    return carry2

                lax.fori_loop(0, KC2 // 16, scale, 0)
                pltpu.async_copy(rows[b], slab.at[di[b]], sem_s[b], add=True)
            return carry

        lax.fori_loop(0, NCH2 // 2, pair, 0)
        pltpu.make_async_copy(rows[0], slab.at[di[0]], sem_s[0]).wait()
        pltpu.make_async_copy(rows[1], slab.at[di[1]], sem_s[1]).wait()
        plsc.subcore_barrier()

        def cpout(i, carry, r=r):
            pltpu.sync_copy(
                slab.at[pl.ds(sid * PER_TILE + i * KC2, KC2), :],
                rows0)
            pltpu.sync_copy(
                rows0,
                part_h.at[pl.ds((cid * 4 + r) * NP + sid * PER_TILE + i * KC2,
                                KC2), :])
            return carry

        lax.fori_loop(0, PER_TILE // KC2, cpout, 0)
        if r < 3:
            zero_rows()
            zero_slab()
        plsc.subcore_barrier()


_sc2 = pl.kernel(
    _sc2_body,
    out_type=[jax.ShapeDtypeStruct((EE,), _f32),
              jax.ShapeDtypeStruct((NCORE * 4 * NP, 16), _f32)],
    mesh=_MESH,
    scratch_types=[pltpu.VMEM((KC2,), jnp.int32),
                   pltpu.VMEM((KC2,), jnp.int32),
                   pltpu.VMEM((KC2,), jnp.int32),
                   pltpu.VMEM((KC2,), jnp.int32),
                   pltpu.VMEM((KC2,), _f32),
                   pltpu.VMEM((KC2,), _f32),
                   pltpu.VMEM((KC2,), _f32),
                   pltpu.VMEM((KC2,), _f32),
                   pltpu.VMEM((KC2, 16), _f32),
                   pltpu.VMEM((KC2, 16), _f32),
                   pltpu.VMEM_SHARED((NP, 16), _f32),
                   pltpu.SemaphoreType.DMA,
                   pltpu.SemaphoreType.DMA,
                   pltpu.SemaphoreType.DMA,
                   pltpu.SemaphoreType.DMA,
                   pltpu.SemaphoreType.DMA,
                   pltpu.SemaphoreType.DMA],
    compiler_params=pltpu.CompilerParams(use_tc_tiling_on_sc=False),
)


def kernel(x, h, edge_attr, edge_index, mlp_x_w0, mlp_x_b0, mlp_x_w1, mlp_x_b1,
           mlp_h_w0, mlp_h_b0, mlp_h_w1, mlp_h_b1,
           gat_W, gat_att_src, gat_att_dst, gat_lin_edge, gat_att_edge, gat_bias,
           fc_mu_w0, fc_mu_b0, fc_mu_w1, fc_mu_b1):
    src = edge_index[0]
    dst = edge_index[1]
    ea = edge_attr[:, 0]
    row = lambda v: v.reshape(1, -1)

    stats = _ea_stats(ea)
    oinit = _encode(x, h, mlp_x_w0, row(mlp_x_b0), mlp_x_w1, row(mlp_x_b1),
                    mlp_h_w0, row(mlp_h_b0), mlp_h_w1, row(mlp_h_b1))
    part = None
    for ii in range(NLAYER):
        xl, x0, x1, x2, x3, a3, b3, scal = _prep(
            oinit, part, gat_W[ii], row(gat_att_src[ii]), row(gat_att_dst[ii]),
            gat_lin_edge[ii], row(gat_att_edge[ii]), stats)
        ex, dpart = _sc1(src, dst, ea, a3.reshape(NN), b3.reshape(NN), scal)
        dpt = (dpart.reshape(NCORE, NP)[:, :NN]
               .reshape(NCORE, GRID, RB).transpose(1, 0, 2))
        oinit, rd3 = _combine(dpt, a3, b3, xl, row(gat_bias[ii]), scal)
        _, part = _sc2(src, dst, ex, rd3.reshape(NN), x0, x1, x2, x3)

    mu = _decode(oinit, part, fc_mu_w0, row(fc_mu_b0), fc_mu_w1, row(fc_mu_b1))
    return mu.reshape(NN, 8, 3)


# speedup vs baseline: 30.5466x; 1.0000x over previous
import functools

import jax
import jax.numpy as jnp
from jax import lax
from jax.experimental import pallas as pl
from jax.experimental.pallas import tpu as pltpu
from jax.experimental.pallas import tpu_sc as plsc

NN = 100000
EE = 1600000
DD = 64
NLAYER = 3

NCORE = 2
NSUB = 16
PER_TILE = 6400
NP = NSUB * PER_TILE
ET = EE // (NCORE * NSUB)
KC = 2000
NCH = ET // KC
KC2 = 400
NCH2 = ET // KC2
RB = 800
GRID = NN // RB

_f32 = jnp.float32


def _ea_stats_body(ea_ref, out_ref):
    blk = ea_ref[...]
    out_ref[0] = jnp.sum(blk)
    out_ref[1] = jnp.max(blk)
    out_ref[2] = jnp.min(blk)
    out_ref[3] = 0.0


def _ea_stats(ea):
    return pl.pallas_call(
        _ea_stats_body,
        out_shape=jax.ShapeDtypeStruct((4,), _f32),
        out_specs=pl.BlockSpec(memory_space=pltpu.SMEM),
    )(ea.reshape(3125, 512))


def _encode_body(x_ref, h_ref, xw0, xb0, xw1, xb1, hw0, hb0, hw1, hb1, o_ref):
    silu = jax.nn.silu
    xl = silu(jnp.dot(x_ref[...], xw0[...], preferred_element_type=_f32)
              + xb0[...]) @ xw1[...] + xb1[...]
    hl = silu(jnp.dot(h_ref[...], hw0[...], preferred_element_type=_f32)
              + hb0[...]) @ hw1[...] + hb1[...]
    o_ref[...] = jnp.concatenate([xl, hl], axis=1)


def _encode(x, h, xw0, xb0, xw1, xb1, hw0, hb0, hw1, hb1):
    R = 2000
    full = lambda a: pl.BlockSpec(a.shape, lambda i: (0,) * a.ndim)
    return pl.pallas_call(
        _encode_body,
        grid=(NN // R,),
        in_specs=[pl.BlockSpec((R, 3), lambda i: (i, 0)),
                  pl.BlockSpec((R, 5), lambda i: (i, 0)),
                  full(xw0), full(xb0), full(xw1), full(xb1),
                  full(hw0), full(hb0), full(hw1), full(hb1)],
        out_specs=pl.BlockSpec((R, DD), lambda i: (i, 0)),
        out_shape=jax.ShapeDtypeStruct((NN, DD), _f32),
    )(x, h, xw0, xb0, xw1, xb1, hw0, hb0, hw1, hb1)


def _prep_body(nparts, *refs):
    it = iter(refs)
    if nparts:
        oinit = next(it)
        parts = [next(it) for _ in range(nparts)]
    else:
        inp_ref = next(it)
    w_ref, asrc_ref, adst_ref, lin_ref, ae_ref, stats_ref = (next(it) for _ in range(6))
    xl_ref, x0_ref, x1_ref, x2_ref, x3_ref, a3_ref, b3_ref, scal_ref = (
        next(it) for _ in range(8))
    acc = next(it)

    i = pl.program_id(0)
    if nparts:
        cols = []
        for r in range(4):
            col = parts[r][...]
            for c in range(1, NCORE):
                col = col + parts[c * 4 + r][...]
            cols.append(col)
        inpb = oinit[...] + jnp.concatenate(cols, axis=1)
    else:
        inpb = inp_ref[...]
    xlb = jnp.dot(inpb, w_ref[...], preferred_element_type=_f32)
    xl_ref[...] = xlb
    x0_ref[...] = xlb[:, 0:16]
    x1_ref[...] = xlb[:, 16:32]
    x2_ref[...] = xlb[:, 32:48]
    x3_ref[...] = xlb[:, 48:64]
    asb = jnp.sum(xlb * asrc_ref[...], axis=1)
    adb = jnp.sum(xlb * adst_ref[...], axis=1)
    a3_ref[0, 0, :] = asb
    b3_ref[0, 0, :] = adb
    m1 = jnp.max(asb)
    m2 = jnp.max(adb)

    @pl.when(i == 0)
    def _():
        acc[0] = m1
        acc[1] = m2

    @pl.when(i > 0)
    def _():
        acc[0] = jnp.maximum(acc[0], m1)
        acc[1] = jnp.maximum(acc[1], m2)

    @pl.when(i == GRID - 1)
    def _():
        c = jnp.sum(lin_ref[...] * ae_ref[...])
        eam = stats_ref[0] / EE
        am = jnp.maximum(jnp.maximum(c * stats_ref[1], c * stats_ref[2]), c * eam)
        sraw = acc[0] + acc[1] + am
        scal_ref[0] = c
        scal_ref[1] = jnp.where(sraw >= 0.0, sraw, 0.2 * sraw)
        scal_ref[2] = eam


_NPB = NP // RB


def _prep(oinit, part_flat, W, asrc_row, adst_row, lin_row, ae_row, stats):
    nparts = 0 if part_flat is None else 4 * NCORE
    full = lambda a: pl.BlockSpec(a.shape, lambda i: (0,) * a.ndim)
    in_specs = [pl.BlockSpec((RB, DD), lambda i: (i, 0))]
    args = [oinit]
    if nparts:
        in_specs += [pl.BlockSpec((RB, 16), lambda i, k=k: (k * _NPB + i, 0))
                     for k in range(nparts)]
        args += [part_flat] * nparts
    in_specs += [full(W), full(asrc_row), full(adst_row), full(lin_row),
                 full(ae_row), pl.BlockSpec(memory_space=pltpu.SMEM)]
    args += [W, asrc_row, adst_row, lin_row, ae_row, stats]
    out_specs = [
        pl.BlockSpec((RB, DD), lambda i: (i, 0)),
        pl.BlockSpec((RB, 16), lambda i: (i, 0)),
        pl.BlockSpec((RB, 16), lambda i: (i, 0)),
        pl.BlockSpec((RB, 16), lambda i: (i, 0)),
        pl.BlockSpec((RB, 16), lambda i: (i, 0)),
        pl.BlockSpec((1, 1, RB), lambda i: (i, 0, 0)),
        pl.BlockSpec((1, 1, RB), lambda i: (i, 0, 0)),
        pl.BlockSpec(memory_space=pltpu.SMEM),
    ]
    out_shape = [
        jax.ShapeDtypeStruct((NN, DD), _f32),
        jax.ShapeDtypeStruct((NN, 16), _f32),
        jax.ShapeDtypeStruct((NN, 16), _f32),
        jax.ShapeDtypeStruct((NN, 16), _f32),
        jax.ShapeDtypeStruct((NN, 16), _f32),
        jax.ShapeDtypeStruct((GRID, 1, RB), _f32),
        jax.ShapeDtypeStruct((GRID, 1, RB), _f32),
        jax.ShapeDtypeStruct((16,), _f32),
    ]
    return pl.pallas_call(
        functools.partial(_prep_body, nparts),
        grid=(GRID,),
        in_specs=in_specs,
        out_specs=out_specs,
        out_shape=out_shape,
        scratch_shapes=[pltpu.SMEM((4,), _f32)],
    )(*args)


def _combine_body(dpt_ref, a3_ref, b3_ref, xl_ref, bias_ref, scal_ref,
                  oinit_ref, rd3_ref):
    p = dpt_ref[0]
    de = p[0, :]
    for c in range(1, NCORE):
        de = de + p[c, :]
    asb = a3_ref[0, 0, :]
    adb = b3_ref[0, 0, :]
    c = scal_ref[0]
    S = scal_ref[1]
    eam = scal_ref[2]
    t = asb + adb + c * eam
    t = jnp.where(t >= 0.0, t, 0.2 * t)
    exs = jnp.exp(t - S)
    rden = 1.0 / (de + exs + 1e-16)
    rd3_ref[0, 0, :] = rden
    cs = exs * rden
    oinit_ref[...] = cs[:, None] * xl_ref[...] + bias_ref[...]


def _combine(dpt, a3, b3, xl, bias_row, scal):
    return pl.pallas_call(
        _combine_body,
        grid=(GRID,),
        in_specs=[pl.BlockSpec((1, NCORE, RB), lambda i: (i, 0, 0)),
                  pl.BlockSpec((1, 1, RB), lambda i: (i, 0, 0)),
                  pl.BlockSpec((1, 1, RB), lambda i: (i, 0, 0)),
                  pl.BlockSpec((RB, DD), lambda i: (i, 0)),
                  pl.BlockSpec((1, DD), lambda i: (0, 0)),
                  pl.BlockSpec(memory_space=pltpu.SMEM)],
        out_specs=[pl.BlockSpec((RB, DD), lambda i: (i, 0)),
                   pl.BlockSpec((1, 1, RB), lambda i: (i, 0, 0))],
        out_shape=[jax.ShapeDtypeStruct((NN, DD), _f32),
                   jax.ShapeDtypeStruct((GRID, 1, RB), _f32)],
    )(dpt, a3, b3, xl, bias_row, scal)


def _decode_body(oinit_ref, *refs):
    parts = refs[:4 * NCORE]
    w0_ref, b0_ref, w1_ref, b1_ref, o_ref = refs[4 * NCORE:]
    cols = []
    for r in range(4):
        col = parts[r][...]
        for c in range(1, NCORE):
            col = col + parts[c * 4 + r][...]
        cols.append(col)
    inpb = oinit_ref[...] + jnp.concatenate(cols, axis=1)
    hh = jax.nn.silu(jnp.dot(inpb, w0_ref[...], preferred_element_type=_f32)
                     + b0_ref[...])
    o_ref[...] = jnp.dot(hh, w1_ref[...], preferred_element_type=_f32) + b1_ref[...]


def _decode(oinit, part_flat, w0, b0_row, w1, b1_row):
    full = lambda a: pl.BlockSpec(a.shape, lambda i: (0,) * a.ndim)
    return pl.pallas_call(
        _decode_body,
        grid=(GRID,),
        in_specs=[pl.BlockSpec((RB, DD), lambda i: (i, 0))]
                 + [pl.BlockSpec((RB, 16), lambda i, k=k: (k * _NPB + i, 0))
                    for k in range(4 * NCORE)]
                 + [full(w0), full(b0_row), full(w1), full(b1_row)],
        out_specs=pl.BlockSpec((RB, 24), lambda i: (i, 0)),
        out_shape=jax.ShapeDtypeStruct((NN, 24), _f32),
    )(oinit, *([part_flat] * (4 * NCORE)), w0, b0_row, w1, b1_row)


_MESH = plsc.VectorSubcoreMesh(core_axis_name="c", subcore_axis_name="s",
                               num_cores=NCORE, num_subcores=16)


def _sc1_body(src_h, dst_h, ea_h, asrc_h, adst_h, scal_h,
              ex_h, dpart_h,
              si0, si1, di0, di1, ea0, ea1, ex0, ex1, asg, adg, zb, scv, dslab,
              sem_i0, sem_i1, sem_g, sem_w0, sem_w1, sem_s0, sem_s1):
    cid = lax.axis_index("c")
    sid = lax.axis_index("s")
    si = (si0, si1)
    di = (di0, di1)
    eab = (ea0, ea1)
    exb = (ex0, ex1)
    sem_i = (sem_i0, sem_i1)
    sem_w = (sem_w0, sem_w1)
    sem_s = (sem_s0, sem_s1)
    pltpu.sync_copy(scal_h, scv)
    sv = scv[pl.ds(0, 16)]
    c = sv[0]
    S = sv[1]

    def zfill(i, carry):
        zb[pl.ds(i * 16, 16)] = jnp.zeros((16,), _f32)
        return carry

    lax.fori_loop(0, PER_TILE // 16, zfill, 0)
    pltpu.sync_copy(zb, dslab.at[pl.ds(sid * PER_TILE, PER_TILE)])
    plsc.subcore_barrier()

    base_e = cid * (EE // NCORE) + sid * ET

    def issue_idx(j, b):
        off = base_e + j * KC
        pltpu.async_copy(src_h.at[pl.ds(off, KC)], si[b], sem_i[b])
        pltpu.async_copy(dst_h.at[pl.ds(off, KC)], di[b], sem_i[b])
        pltpu.async_copy(ea_h.at[pl.ds(off, KC)], eab[b], sem_i[b])

    def wait_idx(j, b):
        off = base_e + j * KC
        pltpu.make_async_copy(src_h.at[pl.ds(off, KC)], si[b], sem_i[b]).wait()
        pltpu.make_async_copy(dst_h.at[pl.ds(off, KC)], di[b], sem_i[b]).wait()
        pltpu.make_async_copy(ea_h.at[pl.ds(off, KC)], eab[b], sem_i[b]).wait()

    issue_idx(0, 0)

    def do_chunk(j, b):
        off = base_e + j * KC
        wait_idx(j, b)
        g1 = pltpu.async_copy(asrc_h.at[si[b]], asg, sem_g)
        g2 = pltpu.async_copy(adst_h.at[di[b]], adg, sem_g)

        @pl.when(j >= 1)
        def _():
            ob = 1 - b
            oo = base_e + (j - 1) * KC
            pltpu.make_async_copy(exb[ob], ex_h.at[pl.ds(oo, KC)],
                                  sem_w[ob]).wait()
            pltpu.make_async_copy(exb[ob], dslab.at[di[ob]],
                                  sem_s[ob]).wait()

        @pl.when(j + 1 < NCH)
        def _():
            issue_idx(j + 1, 1 - b)

        g1.wait()
        g2.wait()

        def comp(t, carry2):
            sl = pl.ds(t * 16, 16)
            av = asg[sl] + adg[sl] + c * eab[b][sl]
            av = jnp.where(av >= 0.0, av, 0.2 * av) - S
            exb[b][sl] = jnp.exp(av)
            return carry2

        lax.fori_loop(0, KC // 16, comp, 0)
        pltpu.async_copy(exb[b], ex_h.at[pl.ds(off, KC)], sem_w[b])
        pltpu.async_copy(exb[b], dslab.at[di[b]], sem_s[b], add=True)

    def pair(jj, carry):
        do_chunk(2 * jj, 0)
        do_chunk(2 * jj + 1, 1)
        return carry

    lax.fori_loop(0, NCH // 2, pair, 0)
    if NCH % 2:
        do_chunk(NCH - 1, 0)
    lb = (NCH - 1) % 2
    pltpu.make_async_copy(exb[lb], ex_h.at[pl.ds(base_e + (NCH - 1) * KC, KC)],
                          sem_w[lb]).wait()
    pltpu.make_async_copy(exb[lb], dslab.at[di[lb]], sem_s[lb]).wait()
    plsc.subcore_barrier()
    pltpu.sync_copy(dslab.at[pl.ds(sid * PER_TILE, PER_TILE)], zb)
    pltpu.sync_copy(zb, dpart_h.at[pl.ds(cid * NP + sid * PER_TILE, PER_TILE)])


_sc1 = pl.kernel(
    _sc1_body,
    out_type=[jax.ShapeDtypeStruct((EE,), _f32),
              jax.ShapeDtypeStruct((NCORE * NP,), _f32)],
    mesh=_MESH,
    scratch_types=[pltpu.VMEM((KC,), jnp.int32),
                   pltpu.VMEM((KC,), jnp.int32),
                   pltpu.VMEM((KC,), jnp.int32),
                   pltpu.VMEM((KC,), jnp.int32),
                   pltpu.VMEM((KC,), _f32),
                   pltpu.VMEM((KC,), _f32),
                   pltpu.VMEM((KC,), _f32),
                   pltpu.VMEM((KC,), _f32),
                   pltpu.VMEM((KC,), _f32),
                   pltpu.VMEM((KC,), _f32),
                   pltpu.VMEM((PER_TILE,), _f32),
                   pltpu.VMEM((16,), _f32),
                   pltpu.VMEM_SHARED((NP,), _f32),
                   pltpu.SemaphoreType.DMA,
                   pltpu.SemaphoreType.DMA,
                   pltpu.SemaphoreType.DMA,
                   pltpu.SemaphoreType.DMA,
                   pltpu.SemaphoreType.DMA,
                   pltpu.SemaphoreType.DMA,
                   pltpu.SemaphoreType.DMA],
    compiler_params=pltpu.CompilerParams(use_tc_tiling_on_sc=False),
)


def _sc2_body(src_h, dst_h, ex_h, rd_h, x0_h, x1_h, x2_h, x3_h,
              coef_h, part_h,
              si0, si1, di0, di1, cfb0, cfb1, exb, rdg,
              rows0, rows1, slab,
              sem_i0, sem_i1, sem_g, sem_s0, sem_s1, sem_r):
    cid = lax.axis_index("c")
    sid = lax.axis_index("s")
    si = (si0, si1)
    di = (di0, di1)
    cfb = (cfb0, cfb1)
    rows = (rows0, rows1)
    sem_i = (sem_i0, sem_i1)
    sem_s = (sem_s0, sem_s1)

    def zero_rows():
        def zf(i, carry):
            rows0[i, :] = jnp.zeros((16,), _f32)
            return carry
        lax.fori_loop(0, KC2, zf, 0)

    def zero_slab():
        def zc(i, carry):
            pltpu.sync_copy(rows0,
                            slab.at[pl.ds(sid * PER_TILE + i * KC2, KC2), :])
            return carry
        lax.fori_loop(0, PER_TILE // KC2, zc, 0)

    base_e = cid * (EE // NCORE) + sid * ET
    zero_rows()
    zero_slab()
    plsc.subcore_barrier()

    for r in range(4):
        xr_h = (x0_h, x1_h, x2_h, x3_h)[r]

        def issue_idx(j, b, r=r):
            off = base_e + j * KC2
            pltpu.async_copy(src_h.at[pl.ds(off, KC2)], si[b], sem_i[b])
            if r == 0:
                pltpu.async_copy(ex_h.at[pl.ds(off, KC2)], exb, sem_i[b])
            else:
                pltpu.async_copy(coef_h.at[pl.ds(off, KC2)], cfb[b], sem_i[b])

        def wait_idx(j, b, r=r):
            off = base_e + j * KC2
            pltpu.make_async_copy(src_h.at[pl.ds(off, KC2)], si[b],
                                  sem_i[b]).wait()
            if r == 0:
                pltpu.make_async_copy(ex_h.at[pl.ds(off, KC2)], exb,
                                      sem_i[b]).wait()
            else:
                pltpu.make_async_copy(coef_h.at[pl.ds(off, KC2)], cfb[b],
                                      sem_i[b]).wait()

        issue_idx(0, 0)

        def do_chunk(j, b, r=r, xr_h=xr_h):
            off = base_e + j * KC2

            @pl.when(j >= 2)
            def _():
                pltpu.make_async_copy(rows[b], slab.at[di[b]],
                                      sem_s[b]).wait()

            wait_idx(j, b)
            g = pltpu.async_copy(xr_h.at[si[b]], rows[b], sem_g)
            pltpu.sync_copy(dst_h.at[pl.ds(off, KC2)], di[b])
            if r == 0:
                pltpu.async_copy(rd_h.at[di[b]], rdg, sem_r).wait()

                def cf(t, carry2):
                    sl = pl.ds(t * 16, 16)
                    cfb[b][sl] = exb[sl] * rdg[sl]
                    return carry2

                lax.fori_loop(0, KC2 // 16, cf, 0)
                pltpu.sync_copy(cfb[b], coef_h.at[pl.ds(off, KC2)])

            @pl.when(j + 1 < NCH2)
            def _():
                issue_idx(j + 1, 1 - b)

            g.wait()

            def scale(t, carry2):
                cv = cfb[b][pl.ds(t * 16, 16)]
                for u in range(16):
                    rows[b][t * 16 + u, :] = rows[b][t * 16 + u, :] * cv[u]
                return carry2

            lax.fori_loop(0, KC2 // 16, scale, 0)
            pltpu.async_copy(rows[b], slab.at[di[b]], sem_s[b], add=True)

        def pair(jj, carry):
            do_chunk(2 * jj, 0)
            do_chunk(2 * jj + 1, 1)
            return carry

        lax.fori_loop(0, NCH2 // 2, pair, 0)
        if NCH2 % 2:
            do_chunk(NCH2 - 1, 0)
        pltpu.make_async_copy(rows[0], slab.at[di[0]], sem_s[0]).wait()
        pltpu.make_async_copy(rows[1], slab.at[di[1]], sem_s[1]).wait()
        plsc.subcore_barrier()

        def cpout(i, carry, r=r):
            pltpu.sync_copy(
                slab.at[pl.ds(sid * PER_TILE + i * KC2, KC2), :],
                rows0)
            pltpu.sync_copy(
                rows0,
                part_h.at[pl.ds((cid * 4 + r) * NP + sid * PER_TILE + i * KC2,
                                KC2), :])
            return carry

        lax.fori_loop(0, PER_TILE // KC2, cpout, 0)
        if r < 3:
            zero_rows()
            zero_slab()
        plsc.subcore_barrier()


_sc2 = pl.kernel(
    _sc2_body,
    out_type=[jax.ShapeDtypeStruct((EE,), _f32),
              jax.ShapeDtypeStruct((NCORE * 4 * NP, 16), _f32)],
    mesh=_MESH,
    scratch_types=[pltpu.VMEM((KC2,), jnp.int32),
                   pltpu.VMEM((KC2,), jnp.int32),
                   pltpu.VMEM((KC2,), jnp.int32),
                   pltpu.VMEM((KC2,), jnp.int32),
                   pltpu.VMEM((KC2,), _f32),
                   pltpu.VMEM((KC2,), _f32),
                   pltpu.VMEM((KC2,), _f32),
                   pltpu.VMEM((KC2,), _f32),
                   pltpu.VMEM((KC2, 16), _f32),
                   pltpu.VMEM((KC2, 16), _f32),
                   pltpu.VMEM_SHARED((NP, 16), _f32),
                   pltpu.SemaphoreType.DMA,
                   pltpu.SemaphoreType.DMA,
                   pltpu.SemaphoreType.DMA,
                   pltpu.SemaphoreType.DMA,
                   pltpu.SemaphoreType.DMA,
                   pltpu.SemaphoreType.DMA],
    compiler_params=pltpu.CompilerParams(use_tc_tiling_on_sc=False),
)


def kernel(x, h, edge_attr, edge_index, mlp_x_w0, mlp_x_b0, mlp_x_w1, mlp_x_b1,
           mlp_h_w0, mlp_h_b0, mlp_h_w1, mlp_h_b1,
           gat_W, gat_att_src, gat_att_dst, gat_lin_edge, gat_att_edge, gat_bias,
           fc_mu_w0, fc_mu_b0, fc_mu_w1, fc_mu_b1):
    src = edge_index[0]
    dst = edge_index[1]
    ea = edge_attr[:, 0]
    row = lambda v: v.reshape(1, -1)

    stats = _ea_stats(ea)
    oinit = _encode(x, h, mlp_x_w0, row(mlp_x_b0), mlp_x_w1, row(mlp_x_b1),
                    mlp_h_w0, row(mlp_h_b0), mlp_h_w1, row(mlp_h_b1))
    part = None
    for ii in range(NLAYER):
        xl, x0, x1, x2, x3, a3, b3, scal = _prep(
            oinit, part, gat_W[ii], row(gat_att_src[ii]), row(gat_att_dst[ii]),
            gat_lin_edge[ii], row(gat_att_edge[ii]), stats)
        ex, dpart = _sc1(src, dst, ea, a3.reshape(NN), b3.reshape(NN), scal)
        dpt = (dpart.reshape(NCORE, NP)[:, :NN]
               .reshape(NCORE, GRID, RB).transpose(1, 0, 2))
        oinit, rd3 = _combine(dpt, a3, b3, xl, row(gat_bias[ii]), scal)
        _, part = _sc2(src, dst, ex, rd3.reshape(NN), x0, x1, x2, x3)

    mu = _decode(oinit, part, fc_mu_w0, row(fc_mu_b0), fc_mu_w1, row(fc_mu_b1))
    return mu.reshape(NN, 8, 3)
